# baseline (speedup 1.0000x reference)
import functools

import jax
import jax.numpy as jnp
from jax import lax
from jax.experimental import pallas as pl
from jax.experimental.pallas import tpu as pltpu
from jax.experimental.pallas import tpu_sc as plsc

_NC = 2
_NS = 16
_NW = _NC * _NS
_CH = 800


def _gather_body(N, L, D, ids_hbm, tab_hbm, out_hbm,
                 ids_v, buf0, buf1, gsem0, gsem1, ssem0, ssem1):
    wid = lax.axis_index("s") * _NC + lax.axis_index("c")
    per_w = N // _NW
    nch = per_w // _CH
    base = wid * per_w
    bufs = (buf0, buf1)
    gsems = (gsem0, gsem1)
    ssems = (ssem0, ssem1)

    pltpu.sync_copy(ids_hbm.at[pl.ds(base, per_w)], ids_v)

    def gather_desc(c, slot):
        idx = ids_v.at[pl.ds(c * _CH, _CH)]
        return pltpu.make_async_copy(
            tab_hbm.at[idx], bufs[slot], gsems[slot])

    def scatter_desc(c, slot):
        return pltpu.make_async_copy(
            bufs[slot], out_hbm.at[pl.ds(base + c * _CH, _CH), :],
            ssems[slot])

    gather_desc(0, 0).start()
    gather_desc(1, 1).start()

    def chunk_pair(c2, carry):
        for slot in range(2):
            c = 2 * c2 + slot
            gather_desc(c, slot).wait()
            scatter_desc(c, slot).start()

            @pl.when(c2 < nch // 2 - 1)
            def _():
                scatter_desc(c, slot).wait()
                gather_desc(c + 2, slot).start()
        return carry

    lax.fori_loop(0, nch // 2, chunk_pair, 0)
    scatter_desc(nch - 2, 0).wait()
    scatter_desc(nch - 1, 1).wait()


def kernel(input_ids, content_table, pos_table):
    B, L = input_ids.shape
    V, D = content_table.shape
    N = B * L
    assert N % (_NW * _CH) == 0
    ids = input_ids.astype(jnp.int32).reshape(N)

    mesh = plsc.VectorSubcoreMesh(core_axis_name="c", subcore_axis_name="s")
    k = functools.partial(
        pl.kernel,
        out_type=jax.ShapeDtypeStruct((N, D), jnp.float32),
        mesh=mesh,
        compiler_params=pltpu.CompilerParams(use_tc_tiling_on_sc=False),
        scratch_types=[
            pltpu.VMEM((N // _NW,), jnp.int32),
            pltpu.VMEM((_CH, D), jnp.float32),
            pltpu.VMEM((_CH, D), jnp.float32),
            pltpu.SemaphoreType.DMA,
            pltpu.SemaphoreType.DMA,
            pltpu.SemaphoreType.DMA,
            pltpu.SemaphoreType.DMA,
        ],
    )(functools.partial(_gather_body, N, L, D))
    content = k(ids, content_table)
    pos_full = jnp.broadcast_to(pos_table[None], (B, L, D)).reshape(N, D)
    return (content + pos_full).reshape(B, L, D)


# speedup vs baseline: 1.2800x; 1.2800x over previous
import functools

import jax
import jax.numpy as jnp
from jax import lax
from jax.experimental import pallas as pl
from jax.experimental.pallas import tpu as pltpu
from jax.experimental.pallas import tpu_sc as plsc

_NC = 2
_NS = 16
_NW = _NC * _NS
_CH = 800


def _gather_body(N, L, D, ids_hbm, tab_hbm, out_hbm,
                 ids_v, buf0, buf1, gsem0, gsem1, ssem0, ssem1):
    wid = lax.axis_index("s") * _NC + lax.axis_index("c")
    per_w = N // _NW
    nch = per_w // _CH
    base = wid * per_w
    bufs = (buf0, buf1)
    gsems = (gsem0, gsem1)
    ssems = (ssem0, ssem1)

    pltpu.sync_copy(ids_hbm.at[pl.ds(base, per_w)], ids_v)

    def gather_desc(c, slot):
        idx = ids_v.at[pl.ds(c * _CH, _CH)]
        return pltpu.make_async_copy(
            tab_hbm.at[idx], bufs[slot], gsems[slot])

    def scatter_desc(c, slot):
        return pltpu.make_async_copy(
            bufs[slot], out_hbm.at[pl.ds(base + c * _CH, _CH), :],
            ssems[slot])

    gather_desc(0, 0).start()
    gather_desc(1, 1).start()

    def chunk_pair(c2, carry):
        for slot in range(2):
            c = 2 * c2 + slot
            gather_desc(c, slot).wait()
            scatter_desc(c, slot).start()

            @pl.when(c2 < nch // 2 - 1)
            def _():
                scatter_desc(c, slot).wait()
                gather_desc(c + 2, slot).start()
        return carry

    lax.fori_loop(0, nch // 2, chunk_pair, 0)
    scatter_desc(nch - 2, 0).wait()
    scatter_desc(nch - 1, 1).wait()


def kernel(input_ids, content_table, pos_table):
    B, L = input_ids.shape
    V, D = content_table.shape
    N = B * L
    assert N % (_NW * _CH) == 0
    ids = input_ids.astype(jnp.int32).reshape(N)

    mesh = plsc.VectorSubcoreMesh(core_axis_name="c", subcore_axis_name="s")
    k = functools.partial(
        pl.kernel,
        out_type=jax.ShapeDtypeStruct((N, D), jnp.float32),
        mesh=mesh,
        compiler_params=pltpu.CompilerParams(use_tc_tiling_on_sc=False),
        scratch_types=[
            pltpu.VMEM((N // _NW,), jnp.int32),
            pltpu.VMEM((_CH, D), jnp.float32),
            pltpu.VMEM((_CH, D), jnp.float32),
            pltpu.SemaphoreType.DMA,
            pltpu.SemaphoreType.DMA,
            pltpu.SemaphoreType.DMA,
            pltpu.SemaphoreType.DMA,
        ],
    )(functools.partial(_gather_body, N, L, D))
    content = k(ids, content_table)
    return content.reshape(B, L, D) + pos_table[None, :, :]
